# baseline (speedup 1.0000x reference)
import functools

import jax
import jax.numpy as jnp
from jax.experimental import pallas as pl
from jax.experimental.pallas import tpu as pltpu

KH = 5
CR1, WB1, CIN1, COUT1, NWB1 = 300, 60, 3, 8, 5
CR2, WB2, CIN2, COUT2, NWB2 = 150, 30, 8, 16, 5

NFEAT = 16 * 75 * 75
FC1_TK = 11264
FC1_NKB = 8
FC1_KPAD = FC1_NKB * FC1_TK

VMEM_LIMIT = 48 * 1024 * 1024


def _convs_kernel(x_ref, B1e_ref, B1o_ref, b1_ref, B2e_ref, B2o_ref,
                  b2_ref, o_ref, y1_ref):
    x = x_ref[0].astype(jnp.bfloat16)
    y1_ref[...] = jnp.zeros_like(y1_ref)

    kin1 = (WB1 + 4) * CIN1
    no1 = (WB1 // 2) * COUT1
    for b in range(NWB1):
        sl = x[:, b * WB1 * CIN1: b * WB1 * CIN1 + kin1]
        acc_e = jnp.dot(sl[0:CR1], B1e_ref[0],
                        preferred_element_type=jnp.float32)
        acc_o = jnp.dot(sl[0:CR1], B1o_ref[0],
                        preferred_element_type=jnp.float32)
        for kh in range(1, KH):
            acc_e += jnp.dot(sl[kh:kh + CR1], B1e_ref[kh],
                             preferred_element_type=jnp.float32)
            acc_o += jnp.dot(sl[kh:kh + CR1], B1o_ref[kh],
                             preferred_element_type=jnp.float32)
        y = jnp.maximum(jnp.maximum(acc_e, acc_o) + b1_ref[...], 0.0)
        y = y.reshape(CR1 // 2, 2, no1).max(axis=1)
        y1_ref[2:2 + CR2, 2 * COUT1 + b * no1: 2 * COUT1 + (b + 1) * no1] = y

    y1 = y1_ref[...].astype(jnp.bfloat16)
    kin2 = (WB2 + 4) * CIN2
    no2 = (WB2 // 2) * COUT2
    for b in range(NWB2):
        sl = y1[:, b * WB2 * CIN2: b * WB2 * CIN2 + kin2]
        acc_e = jnp.dot(sl[0:CR2], B2e_ref[0],
                        preferred_element_type=jnp.float32)
        acc_o = jnp.dot(sl[0:CR2], B2o_ref[0],
                        preferred_element_type=jnp.float32)
        for kh in range(1, KH):
            acc_e += jnp.dot(sl[kh:kh + CR2], B2e_ref[kh],
                             preferred_element_type=jnp.float32)
            acc_o += jnp.dot(sl[kh:kh + CR2], B2o_ref[kh],
                             preferred_element_type=jnp.float32)
        y = jnp.maximum(jnp.maximum(acc_e, acc_o) + b2_ref[...], 0.0)
        y = y.reshape(CR2 // 2, 2, no2).max(axis=1)
        o_ref[0, :, b * no2:(b + 1) * no2] = y


def _convs(xi, B1e, B1o, b1t, B2e, B2o, b2t):
    N = xi.shape[0]
    return pl.pallas_call(
        _convs_kernel,
        out_shape=jax.ShapeDtypeStruct((N, 75, 1200), jnp.float32),
        grid_spec=pltpu.PrefetchScalarGridSpec(
            num_scalar_prefetch=0,
            grid=(N,),
            in_specs=[
                pl.BlockSpec((1, 304, 912), lambda n: (n, 0, 0)),
                pl.BlockSpec((KH, 192, 240), lambda n: (0, 0, 0)),
                pl.BlockSpec((KH, 192, 240), lambda n: (0, 0, 0)),
                pl.BlockSpec((1, 240), lambda n: (0, 0)),
                pl.BlockSpec((KH, 272, 240), lambda n: (0, 0, 0)),
                pl.BlockSpec((KH, 272, 240), lambda n: (0, 0, 0)),
                pl.BlockSpec((1, 240), lambda n: (0, 0)),
            ],
            out_specs=pl.BlockSpec((1, 75, 1200), lambda n: (n, 0, 0)),
            scratch_shapes=[pltpu.VMEM((154, 1232), jnp.float32)],
        ),
        compiler_params=pltpu.CompilerParams(
            dimension_semantics=("parallel",),
            vmem_limit_bytes=VMEM_LIMIT,
        ),
    )(xi, B1e, B1o, b1t, B2e, B2o, b2t)


def _fc_kernel(x_ref, w1_ref, b1_ref, w2_ref, b2_ref, w3_ref, b3_ref,
               o_ref, acc_ref):
    k = pl.program_id(0)

    @pl.when(k == 0)
    def _init():
        acc_ref[...] = jnp.zeros(acc_ref.shape, acc_ref.dtype)

    acc_ref[...] += jnp.dot(x_ref[...], w1_ref[...],
                            preferred_element_type=jnp.float32)

    @pl.when(k == pl.num_programs(0) - 1)
    def _head():
        h = jnp.maximum(acc_ref[...] + b1_ref[...], 0.0)
        h = jnp.maximum(jnp.dot(h, w2_ref[...],
                                preferred_element_type=jnp.float32)
                        + b2_ref[...], 0.0)
        o_ref[...] = (jnp.dot(h, w3_ref[...],
                              preferred_element_type=jnp.float32)
                      + b3_ref[...])


def _fc(feats, wf1, bf1, wf2, bf2, wf3, bf3):
    N, K = feats.shape
    F1 = wf1.shape[1]
    F2 = wf2.shape[1]
    FO = wf3.shape[1]
    return pl.pallas_call(
        _fc_kernel,
        out_shape=jax.ShapeDtypeStruct((N, FO), jnp.float32),
        grid_spec=pltpu.PrefetchScalarGridSpec(
            num_scalar_prefetch=0,
            grid=(FC1_NKB,),
            in_specs=[
                pl.BlockSpec((N, FC1_TK), lambda k: (0, k)),
                pl.BlockSpec((FC1_TK, F1), lambda k: (k, 0)),
                pl.BlockSpec((1, F1), lambda k: (0, 0)),
                pl.BlockSpec((F1, F2), lambda k: (0, 0)),
                pl.BlockSpec((1, F2), lambda k: (0, 0)),
                pl.BlockSpec((F2, FO), lambda k: (0, 0)),
                pl.BlockSpec((1, FO), lambda k: (0, 0)),
            ],
            out_specs=pl.BlockSpec((N, FO), lambda k: (0, 0)),
            scratch_shapes=[pltpu.VMEM((N, F1), jnp.float32)],
        ),
        compiler_params=pltpu.CompilerParams(
            dimension_semantics=("arbitrary",),
            vmem_limit_bytes=VMEM_LIMIT,
        ),
    )(feats, wf1, bf1.reshape(1, -1), wf2, bf2.reshape(1, -1),
      wf3, bf3.reshape(1, -1))


def _split_even_odd(B, w, cout):
    kin = B.shape[1]
    B4 = B.reshape(KH, kin, w, cout)
    Be = B4[:, :, 0::2, :].reshape(KH, kin, (w // 2) * cout)
    Bo = B4[:, :, 1::2, :].reshape(KH, kin, (w // 2) * cout)
    return Be.astype(jnp.bfloat16), Bo.astype(jnp.bfloat16)


def kernel(x, B1, b1t, B2, b2t, wf1, bf1, wf2, bf2, wf3, bf3):
    N = x.shape[0]
    xi = jnp.transpose(x, (0, 2, 3, 1))
    xi = jnp.pad(xi, ((0, 0), (2, 2), (2, 2), (0, 0)))
    xi = xi.reshape(N, 304, 912)
    B1e, B1o = _split_even_odd(B1, WB1, COUT1)
    B2e, B2o = _split_even_odd(B2, WB2, COUT2)
    feats = _convs(xi, B1e, B1o, b1t[:, :(WB1 // 2) * COUT1],
                   B2e, B2o, b2t[:, :(WB2 // 2) * COUT2])
    feats = feats.reshape(N, NFEAT)
    feats = jnp.pad(feats, ((0, 0), (0, FC1_KPAD - NFEAT)))
    return _fc(feats, wf1, bf1, wf2, bf2, wf3, bf3)


# speedup vs baseline: 1.6014x; 1.6014x over previous
import functools

import jax
import jax.numpy as jnp
from jax.experimental import pallas as pl
from jax.experimental.pallas import tpu as pltpu

KH = 5
CR1, WB1, CIN1, COUT1, NWB1 = 300, 60, 3, 8, 5
CR2, WB2, CIN2, COUT2, NWB2 = 150, 30, 8, 16, 5

NFEAT = 16 * 75 * 75
FC1_TK = 11264
FC1_NKB = 8
FC1_KPAD = FC1_NKB * FC1_TK

VMEM_LIMIT = 48 * 1024 * 1024


def _convs_kernel(x_ref, B1e_ref, B1o_ref, b1_ref, B2e_ref, B2o_ref,
                  b2_ref, o_ref, xp_ref, y1_ref):
    xp_ref[...] = jnp.zeros_like(xp_ref)
    xp_ref[:, 2:302, 2:302] = x_ref[0].astype(jnp.bfloat16)
    y1_ref[...] = jnp.zeros_like(y1_ref)

    kin1 = (WB1 + 4) * CIN1
    no1 = (WB1 // 2) * COUT1
    for b in range(NWB1):
        sl = jnp.concatenate(
            [xp_ref[ci, :, b * WB1: b * WB1 + WB1 + 4] for ci in range(CIN1)],
            axis=1)
        acc_e = jnp.dot(sl[0:CR1], B1e_ref[0],
                        preferred_element_type=jnp.float32)
        acc_o = jnp.dot(sl[0:CR1], B1o_ref[0],
                        preferred_element_type=jnp.float32)
        for kh in range(1, KH):
            acc_e += jnp.dot(sl[kh:kh + CR1], B1e_ref[kh],
                             preferred_element_type=jnp.float32)
            acc_o += jnp.dot(sl[kh:kh + CR1], B1o_ref[kh],
                             preferred_element_type=jnp.float32)
        y = jnp.maximum(jnp.maximum(acc_e, acc_o) + b1_ref[...], 0.0)
        y = y.reshape(CR1 // 2, 2, no1).max(axis=1)
        y1_ref[2:2 + CR2, 2 * COUT1 + b * no1: 2 * COUT1 + (b + 1) * no1] = y

    y1 = y1_ref[...].astype(jnp.bfloat16)
    kin2 = (WB2 + 4) * CIN2
    no2 = (WB2 // 2) * COUT2
    for b in range(NWB2):
        sl = y1[:, b * WB2 * CIN2: b * WB2 * CIN2 + kin2]
        acc_e = jnp.dot(sl[0:CR2], B2e_ref[0],
                        preferred_element_type=jnp.float32)
        acc_o = jnp.dot(sl[0:CR2], B2o_ref[0],
                        preferred_element_type=jnp.float32)
        for kh in range(1, KH):
            acc_e += jnp.dot(sl[kh:kh + CR2], B2e_ref[kh],
                             preferred_element_type=jnp.float32)
            acc_o += jnp.dot(sl[kh:kh + CR2], B2o_ref[kh],
                             preferred_element_type=jnp.float32)
        y = jnp.maximum(jnp.maximum(acc_e, acc_o) + b2_ref[...], 0.0)
        y = y.reshape(CR2 // 2, 2, no2).max(axis=1)
        o_ref[0, :, b * no2:(b + 1) * no2] = y


def _convs(xi, B1e, B1o, b1t, B2e, B2o, b2t):
    N = xi.shape[0]
    return pl.pallas_call(
        _convs_kernel,
        out_shape=jax.ShapeDtypeStruct((N, 75, 1200), jnp.float32),
        grid_spec=pltpu.PrefetchScalarGridSpec(
            num_scalar_prefetch=0,
            grid=(N,),
            in_specs=[
                pl.BlockSpec((1, 3, 300, 300), lambda n: (n, 0, 0, 0)),
                pl.BlockSpec((KH, 192, 240), lambda n: (0, 0, 0)),
                pl.BlockSpec((KH, 192, 240), lambda n: (0, 0, 0)),
                pl.BlockSpec((1, 240), lambda n: (0, 0)),
                pl.BlockSpec((KH, 272, 240), lambda n: (0, 0, 0)),
                pl.BlockSpec((KH, 272, 240), lambda n: (0, 0, 0)),
                pl.BlockSpec((1, 240), lambda n: (0, 0)),
            ],
            out_specs=pl.BlockSpec((1, 75, 1200), lambda n: (n, 0, 0)),
            scratch_shapes=[pltpu.VMEM((3, 304, 304), jnp.bfloat16),
                            pltpu.VMEM((154, 1232), jnp.float32)],
        ),
        compiler_params=pltpu.CompilerParams(
            dimension_semantics=("parallel",),
            vmem_limit_bytes=VMEM_LIMIT,
        ),
    )(xi, B1e, B1o, b1t, B2e, B2o, b2t)


def _fc_kernel(x_ref, w1_ref, b1_ref, w2_ref, b2_ref, w3_ref, b3_ref,
               o_ref, acc_ref):
    k = pl.program_id(0)

    @pl.when(k == 0)
    def _init():
        acc_ref[...] = jnp.zeros(acc_ref.shape, acc_ref.dtype)

    acc_ref[...] += jnp.dot(x_ref[...], w1_ref[...],
                            preferred_element_type=jnp.float32)

    @pl.when(k == pl.num_programs(0) - 1)
    def _head():
        h = jnp.maximum(acc_ref[...] + b1_ref[...], 0.0)
        h = jnp.maximum(jnp.dot(h, w2_ref[...],
                                preferred_element_type=jnp.float32)
                        + b2_ref[...], 0.0)
        o_ref[...] = (jnp.dot(h, w3_ref[...],
                              preferred_element_type=jnp.float32)
                      + b3_ref[...])


def _fc(feats, wf1, bf1, wf2, bf2, wf3, bf3):
    N, K = feats.shape
    F1 = wf1.shape[1]
    F2 = wf2.shape[1]
    FO = wf3.shape[1]
    return pl.pallas_call(
        _fc_kernel,
        out_shape=jax.ShapeDtypeStruct((N, FO), jnp.float32),
        grid_spec=pltpu.PrefetchScalarGridSpec(
            num_scalar_prefetch=0,
            grid=(FC1_NKB,),
            in_specs=[
                pl.BlockSpec((N, FC1_TK), lambda k: (0, k)),
                pl.BlockSpec((FC1_TK, F1), lambda k: (k, 0)),
                pl.BlockSpec((1, F1), lambda k: (0, 0)),
                pl.BlockSpec((F1, F2), lambda k: (0, 0)),
                pl.BlockSpec((1, F2), lambda k: (0, 0)),
                pl.BlockSpec((F2, FO), lambda k: (0, 0)),
                pl.BlockSpec((1, FO), lambda k: (0, 0)),
            ],
            out_specs=pl.BlockSpec((N, FO), lambda k: (0, 0)),
            scratch_shapes=[pltpu.VMEM((N, F1), jnp.float32)],
        ),
        compiler_params=pltpu.CompilerParams(
            dimension_semantics=("arbitrary",),
            vmem_limit_bytes=VMEM_LIMIT,
        ),
    )(feats, wf1, bf1.reshape(1, -1), wf2, bf2.reshape(1, -1),
      wf3, bf3.reshape(1, -1))


def _split_even_odd(B, w, cout):
    kin = B.shape[1]
    B4 = B.reshape(KH, kin, w, cout)
    Be = B4[:, :, 0::2, :].reshape(KH, kin, (w // 2) * cout)
    Bo = B4[:, :, 1::2, :].reshape(KH, kin, (w // 2) * cout)
    return Be.astype(jnp.bfloat16), Bo.astype(jnp.bfloat16)


def kernel(x, B1, b1t, B2, b2t, wf1, bf1, wf2, bf2, wf3, bf3):
    N = x.shape[0]
    kin1 = (WB1 + 4) * CIN1
    B1cb = (B1.reshape(KH, WB1 + 4, CIN1, WB1 * COUT1)
            .transpose(0, 2, 1, 3).reshape(KH, kin1, WB1 * COUT1))
    B1e, B1o = _split_even_odd(B1cb, WB1, COUT1)
    B2e, B2o = _split_even_odd(B2, WB2, COUT2)
    feats = _convs(x, B1e, B1o, b1t[:, :(WB1 // 2) * COUT1],
                   B2e, B2o, b2t[:, :(WB2 // 2) * COUT2])
    feats = feats.reshape(N, NFEAT)
    feats = jnp.pad(feats, ((0, 0), (0, FC1_KPAD - NFEAT)))
    return _fc(feats, wf1, bf1, wf2, bf2, wf3, bf3)


# speedup vs baseline: 1.6115x; 1.0063x over previous
import functools

import jax
import jax.numpy as jnp
from jax.experimental import pallas as pl
from jax.experimental.pallas import tpu as pltpu

KH = 5
CR1, WB1, CIN1, COUT1, NWB1 = 300, 60, 3, 8, 5
CR2, WB2, CIN2, COUT2, NWB2 = 150, 30, 8, 16, 5

NFEAT = 16 * 75 * 75
FC1_TK = 11264
FC1_NKB = 8
FC1_KPAD = FC1_NKB * FC1_TK

VMEM_LIMIT = 48 * 1024 * 1024


def _convs_kernel(x_ref, B1_ref, b1_ref, B2_ref, b2_ref, o_ref,
                  xp_ref, y1_ref):
    xp_ref[...] = jnp.zeros_like(xp_ref)
    xp_ref[:, 2:302, 2:302] = x_ref[0].astype(jnp.bfloat16)
    y1_ref[...] = jnp.zeros_like(y1_ref)

    kin1 = (WB1 + 4) * CIN1
    no1 = (WB1 // 2) * COUT1
    for b in range(NWB1):
        sl = jnp.concatenate(
            [xp_ref[ci, :, b * WB1: b * WB1 + WB1 + 4] for ci in range(CIN1)],
            axis=1)
        acc = jnp.dot(sl[0:CR1], B1_ref[0],
                      preferred_element_type=jnp.float32)
        for kh in range(1, KH):
            acc += jnp.dot(sl[kh:kh + CR1], B1_ref[kh],
                           preferred_element_type=jnp.float32)
        y = jnp.maximum(jnp.maximum(acc[:, :no1], acc[:, no1:]) + b1_ref[...],
                        0.0)
        y = y.reshape(CR1 // 2, 2, no1).max(axis=1)
        y1_ref[2:2 + CR2, 2 * COUT1 + b * no1: 2 * COUT1 + (b + 1) * no1] = y

    y1 = y1_ref[...].astype(jnp.bfloat16)
    kin2 = (WB2 + 4) * CIN2
    no2 = (WB2 // 2) * COUT2
    for b in range(NWB2):
        sl = y1[:, b * WB2 * CIN2: b * WB2 * CIN2 + kin2]
        acc = jnp.dot(sl[0:CR2], B2_ref[0],
                      preferred_element_type=jnp.float32)
        for kh in range(1, KH):
            acc += jnp.dot(sl[kh:kh + CR2], B2_ref[kh],
                           preferred_element_type=jnp.float32)
        y = jnp.maximum(jnp.maximum(acc[:, :no2], acc[:, no2:]) + b2_ref[...],
                        0.0)
        y = y.reshape(CR2 // 2, 2, no2).max(axis=1)
        o_ref[0, :, b * no2:(b + 1) * no2] = y


def _convs(xi, B1c, b1t, B2c, b2t):
    N = xi.shape[0]
    return pl.pallas_call(
        _convs_kernel,
        out_shape=jax.ShapeDtypeStruct((N, 75, 1200), jnp.float32),
        grid_spec=pltpu.PrefetchScalarGridSpec(
            num_scalar_prefetch=0,
            grid=(N,),
            in_specs=[
                pl.BlockSpec((1, 3, 300, 300), lambda n: (n, 0, 0, 0)),
                pl.BlockSpec((KH, 192, 480), lambda n: (0, 0, 0)),
                pl.BlockSpec((1, 240), lambda n: (0, 0)),
                pl.BlockSpec((KH, 272, 480), lambda n: (0, 0, 0)),
                pl.BlockSpec((1, 240), lambda n: (0, 0)),
            ],
            out_specs=pl.BlockSpec((1, 75, 1200), lambda n: (n, 0, 0)),
            scratch_shapes=[pltpu.VMEM((3, 304, 304), jnp.bfloat16),
                            pltpu.VMEM((154, 1232), jnp.float32)],
        ),
        compiler_params=pltpu.CompilerParams(
            dimension_semantics=("parallel",),
            vmem_limit_bytes=VMEM_LIMIT,
        ),
    )(xi, B1c, b1t, B2c, b2t)


def _fc_kernel(x_ref, w1_ref, b1_ref, w2_ref, b2_ref, w3_ref, b3_ref,
               o_ref, acc_ref):
    k = pl.program_id(0)

    @pl.when(k == 0)
    def _init():
        acc_ref[...] = jnp.zeros(acc_ref.shape, acc_ref.dtype)

    acc_ref[...] += jnp.dot(x_ref[...], w1_ref[...],
                            preferred_element_type=jnp.float32)

    @pl.when(k == pl.num_programs(0) - 1)
    def _head():
        h = jnp.maximum(acc_ref[...] + b1_ref[...], 0.0)
        h = jnp.maximum(jnp.dot(h, w2_ref[...],
                                preferred_element_type=jnp.float32)
                        + b2_ref[...], 0.0)
        o_ref[...] = (jnp.dot(h, w3_ref[...],
                              preferred_element_type=jnp.float32)
                      + b3_ref[...])


def _fc(feats, wf1, bf1, wf2, bf2, wf3, bf3):
    N, K = feats.shape
    F1 = wf1.shape[1]
    F2 = wf2.shape[1]
    FO = wf3.shape[1]
    return pl.pallas_call(
        _fc_kernel,
        out_shape=jax.ShapeDtypeStruct((N, FO), jnp.float32),
        grid_spec=pltpu.PrefetchScalarGridSpec(
            num_scalar_prefetch=0,
            grid=(FC1_NKB,),
            in_specs=[
                pl.BlockSpec((N, FC1_TK), lambda k: (0, k)),
                pl.BlockSpec((FC1_TK, F1), lambda k: (k, 0)),
                pl.BlockSpec((1, F1), lambda k: (0, 0)),
                pl.BlockSpec((F1, F2), lambda k: (0, 0)),
                pl.BlockSpec((1, F2), lambda k: (0, 0)),
                pl.BlockSpec((F2, FO), lambda k: (0, 0)),
                pl.BlockSpec((1, FO), lambda k: (0, 0)),
            ],
            out_specs=pl.BlockSpec((N, FO), lambda k: (0, 0)),
            scratch_shapes=[pltpu.VMEM((N, F1), jnp.float32)],
        ),
        compiler_params=pltpu.CompilerParams(
            dimension_semantics=("arbitrary",),
            vmem_limit_bytes=VMEM_LIMIT,
        ),
    )(feats, wf1, bf1.reshape(1, -1), wf2, bf2.reshape(1, -1),
      wf3, bf3.reshape(1, -1))


def _even_odd_cols(B, w, cout):
    kin = B.shape[1]
    B4 = B.reshape(KH, kin, w, cout)
    Be = B4[:, :, 0::2, :].reshape(KH, kin, (w // 2) * cout)
    Bo = B4[:, :, 1::2, :].reshape(KH, kin, (w // 2) * cout)
    return jnp.concatenate([Be, Bo], axis=2).astype(jnp.bfloat16)


def kernel(x, B1, b1t, B2, b2t, wf1, bf1, wf2, bf2, wf3, bf3):
    N = x.shape[0]
    kin1 = (WB1 + 4) * CIN1
    B1cb = (B1.reshape(KH, WB1 + 4, CIN1, WB1 * COUT1)
            .transpose(0, 2, 1, 3).reshape(KH, kin1, WB1 * COUT1))
    B1c = _even_odd_cols(B1cb, WB1, COUT1)
    B2c = _even_odd_cols(B2, WB2, COUT2)
    feats = _convs(x, B1c, b1t[:, :(WB1 // 2) * COUT1],
                   B2c, b2t[:, :(WB2 // 2) * COUT2])
    feats = feats.reshape(N, NFEAT)
    feats = jnp.pad(feats, ((0, 0), (0, FC1_KPAD - NFEAT)))
    return _fc(feats, wf1, bf1, wf2, bf2, wf3, bf3)
